# baseline (speedup 1.0000x reference)
import functools

import jax
import jax.numpy as jnp
from jax import lax
from jax.experimental import pallas as pl
from jax.experimental.pallas import tpu as pltpu
from jax.experimental.pallas import tpu_sc as plsc

D = 128
DA = 144
DH = DA // 2
EC = 128
NC = 2
NS = 16
RING = 4
ROW_BLK = 1000


def _haug_body(x_ref, w_ref, b_ref, out_ref):
    k = pl.program_id(1)
    h = jnp.dot(x_ref[...], w_ref[0], preferred_element_type=jnp.float32)
    h = jnp.maximum(h + b_ref[0], 0.0)
    lane = lax.broadcasted_iota(jnp.int32, (ROW_BLK, DH), 1)
    ones_lane = jnp.logical_and(k == 1, lane >= DH - (DA - D))
    out_ref[0] = jnp.where(ones_lane, 1.0, h)


def _haug(x, mlp_kernel, mlp_bias):
    n = x.shape[0]
    w_aug = jnp.concatenate(
        [mlp_kernel, jnp.zeros((D, DA - D), jnp.float32)], axis=1)
    w_st = w_aug.reshape(D, NC, DH).transpose(1, 0, 2)
    b_st = jnp.concatenate(
        [mlp_bias, jnp.zeros((DA - D,), jnp.float32)]).reshape(NC, 1, DH)
    return pl.pallas_call(
        _haug_body,
        grid=(n // ROW_BLK, NC),
        in_specs=[
            pl.BlockSpec((ROW_BLK, D), lambda i, k: (i, 0)),
            pl.BlockSpec((1, D, DH), lambda i, k: (k, 0, 0)),
            pl.BlockSpec((1, 1, DH), lambda i, k: (k, 0, 0)),
        ],
        out_specs=pl.BlockSpec((1, ROW_BLK, DH), lambda i, k: (k, i, 0)),
        out_shape=jax.ShapeDtypeStruct((NC, n, DH), jnp.float32),
    )(x, w_st, b_st)


def _segment_sum_sc(h2, idx_s, zeros_tbl, n_nodes, t_rows, eps):
    nchunks = eps // EC
    rpt = t_rows // NS
    hpt = n_nodes // NS

    mesh = plsc.VectorSubcoreMesh(core_axis_name="c", subcore_axis_name="s")

    @functools.partial(
        pl.kernel,
        out_type=jax.ShapeDtypeStruct((NC, t_rows, DH), jnp.float32),
        mesh=mesh,
        compiler_params=pltpu.CompilerParams(use_tc_tiling_on_sc=False),
        scratch_types=[
            pltpu.VMEM_SHARED((n_nodes, DH), jnp.float32),
            pltpu.VMEM_SHARED((t_rows, DH), jnp.float32),
            [pltpu.VMEM((2, EC), jnp.int32) for _ in range(RING)],
            [pltpu.VMEM((EC, DH), jnp.float32) for _ in range(RING)],
            [pltpu.SemaphoreType.DMA for _ in range(RING)],
        ],
    )
    def k(h_hbm, idx_hbm, z_hbm, out_hbm, htbl, acc, ibufs, gbufs, gsems):
        c = lax.axis_index("c")
        s = lax.axis_index("s")
        r0 = s * rpt

        def fire(j, b):
            pltpu.sync_copy(idx_hbm.at[s, j], ibufs[b])
            pltpu.async_copy(htbl.at[ibufs[b].at[0]], gbufs[b], gsems[b])

        def drain(j, b):
            pltpu.make_async_copy(
                htbl.at[ibufs[b].at[0]], gbufs[b], gsems[b]).wait()
            pltpu.sync_copy(gbufs[b], acc.at[ibufs[b].at[1]], add=True)

        pltpu.sync_copy(h_hbm.at[c, pl.ds(s * hpt, hpt)],
                        htbl.at[pl.ds(s * hpt, hpt)])
        pltpu.sync_copy(z_hbm.at[pl.ds(r0, rpt)], acc.at[pl.ds(r0, rpt)])
        plsc.subcore_barrier()
        for b in range(RING):
            fire(b, b)

        def body(i, carry):
            for b in range(RING):
                j = i * RING + b
                drain(j, b)
                fire(j + RING, b)
            return carry

        lax.fori_loop(0, nchunks // RING - 1, body, 0)
        for b in range(RING):
            drain(nchunks - RING + b, b)
        plsc.subcore_barrier()

        pltpu.sync_copy(acc.at[pl.ds(r0, rpt)], out_hbm.at[c, pl.ds(r0, rpt)])

    return k(h2, idx_s, zeros_tbl)


def _combine_body(p_ref, x_ref, wn_ref, ws_ref, b_ref, out_ref):
    left = p_ref[0]
    right = p_ref[1]
    cnt = jnp.max(right[:, DH - (DA - D):], axis=1, keepdims=True)
    denom = jnp.where(cnt > 0.0, cnt, 1.0)
    r = jnp.concatenate([left, right[:, :DH - (DA - D)]], axis=1) / denom
    fn = jnp.dot(r, wn_ref[...], preferred_element_type=jnp.float32)
    fx = jnp.dot(x_ref[...], ws_ref[...], preferred_element_type=jnp.float32)
    o = jnp.concatenate([fn, fx], axis=1) + b_ref[...]
    out_ref[...] = jnp.maximum(o, 0.0)


def _combine(partials, x, neighs_kernel, self_kernel, bias):
    n = x.shape[0]
    return pl.pallas_call(
        _combine_body,
        grid=(n // ROW_BLK,),
        in_specs=[
            pl.BlockSpec((NC, ROW_BLK, DH), lambda i: (0, i, 0)),
            pl.BlockSpec((ROW_BLK, D), lambda i: (i, 0)),
            pl.BlockSpec((D, D), lambda i: (0, 0)),
            pl.BlockSpec((D, D), lambda i: (0, 0)),
            pl.BlockSpec((1, 2 * D), lambda i: (0, 0)),
        ],
        out_specs=pl.BlockSpec((ROW_BLK, 2 * D), lambda i: (i, 0)),
        out_shape=jax.ShapeDtypeStruct((n, 2 * D), jnp.float32),
    )(partials, x, neighs_kernel, self_kernel, bias.reshape(1, 2 * D))


def kernel(x, edge_index, edge_weight, mlp_kernel, mlp_bias, neighs_kernel,
           self_kernel, bias):
    del edge_weight
    n_nodes = x.shape[0]
    n_edges = edge_index.shape[1]

    eps = -(-n_edges // (NS * EC * RING)) * EC * RING
    epad = NS * eps - n_edges
    t_rows = -(-(n_nodes + 1) // (NS * 8)) * (NS * 8)

    row = edge_index[0]
    col = edge_index[1]
    if epad:
        row = jnp.concatenate([row, jnp.full((epad,), n_nodes, jnp.int32)])
        col = jnp.concatenate([col, jnp.zeros((epad,), jnp.int32)])
    idx_s = jnp.stack(
        [col.reshape(NS, eps // EC, EC), row.reshape(NS, eps // EC, EC)],
        axis=2)

    h2 = _haug(x, mlp_kernel, mlp_bias)
    zeros_tbl = jnp.zeros((t_rows, DH), jnp.float32)
    partials = _segment_sum_sc(h2, idx_s, zeros_tbl, n_nodes, t_rows, eps)
    return _combine(partials, x, neighs_kernel, self_kernel, bias)


# speedup vs baseline: 1.2291x; 1.2291x over previous
import functools

import jax
import jax.numpy as jnp
from jax import lax
from jax.experimental import pallas as pl
from jax.experimental.pallas import tpu as pltpu
from jax.experimental.pallas import tpu_sc as plsc

D = 128
DA = 144
DH = DA // 2
EC = 128
NC = 2
NS = 16
RING = 4
ROW_BLK = 1000


def _haug_body(x_ref, w_ref, b_ref, out_ref):
    k = pl.program_id(1)
    h = jnp.dot(x_ref[...], w_ref[0], preferred_element_type=jnp.float32)
    h = jnp.maximum(h + b_ref[0], 0.0)
    lane = lax.broadcasted_iota(jnp.int32, (ROW_BLK, DH), 1)
    ones_lane = jnp.logical_and(k == 1, lane >= DH - (DA - D))
    out_ref[0] = jnp.where(ones_lane, 1.0, h)


def _haug(x, mlp_kernel, mlp_bias):
    n = x.shape[0]
    w_aug = jnp.concatenate(
        [mlp_kernel, jnp.zeros((D, DA - D), jnp.float32)], axis=1)
    w_st = w_aug.reshape(D, NC, DH).transpose(1, 0, 2)
    b_st = jnp.concatenate(
        [mlp_bias, jnp.zeros((DA - D,), jnp.float32)]).reshape(NC, 1, DH)
    return pl.pallas_call(
        _haug_body,
        grid=(n // ROW_BLK, NC),
        in_specs=[
            pl.BlockSpec((ROW_BLK, D), lambda i, k: (i, 0)),
            pl.BlockSpec((1, D, DH), lambda i, k: (k, 0, 0)),
            pl.BlockSpec((1, 1, DH), lambda i, k: (k, 0, 0)),
        ],
        out_specs=pl.BlockSpec((1, ROW_BLK, DH), lambda i, k: (k, i, 0)),
        out_shape=jax.ShapeDtypeStruct((NC, n, DH), jnp.float32),
    )(x, w_st, b_st)


def _segment_sum_sc(h2, idx_s, zeros_tbl, n_nodes, t_rows, eps):
    nchunks = eps // EC
    rpt = t_rows // NS
    hpt = n_nodes // NS

    mesh = plsc.VectorSubcoreMesh(core_axis_name="c", subcore_axis_name="s")

    @functools.partial(
        pl.kernel,
        out_type=jax.ShapeDtypeStruct((NC, t_rows, DH), jnp.float32),
        mesh=mesh,
        compiler_params=pltpu.CompilerParams(use_tc_tiling_on_sc=False),
        scratch_types=[
            pltpu.VMEM_SHARED((n_nodes, DH), jnp.float32),
            pltpu.VMEM_SHARED((t_rows, DH), jnp.float32),
            [pltpu.VMEM((2, EC), jnp.int32) for _ in range(RING)],
            [pltpu.VMEM((EC, DH), jnp.float32) for _ in range(RING)],
            [pltpu.SemaphoreType.DMA for _ in range(RING)],
            [pltpu.SemaphoreType.DMA for _ in range(RING)],
        ],
    )
    def k(h_hbm, idx_hbm, z_hbm, out_hbm, htbl, acc, ibufs, gbufs, isems,
          gsems):
        c = lax.axis_index("c")
        s = lax.axis_index("s")
        r0 = s * rpt

        def fire_idx(j, q):
            pltpu.async_copy(idx_hbm.at[s, j], ibufs[q], isems[q])

        def wait_idx(j, q):
            pltpu.make_async_copy(idx_hbm.at[s, j], ibufs[q], isems[q]).wait()

        def fire_gather(j, b):
            pltpu.async_copy(htbl.at[ibufs[b].at[0]], gbufs[b], gsems[b])

        def wait_gather(j, b):
            pltpu.make_async_copy(
                htbl.at[ibufs[b].at[0]], gbufs[b], gsems[b]).wait()

        pltpu.sync_copy(h_hbm.at[c, pl.ds(s * hpt, hpt)],
                        htbl.at[pl.ds(s * hpt, hpt)])
        pltpu.sync_copy(z_hbm.at[pl.ds(r0, rpt)], acc.at[pl.ds(r0, rpt)])
        plsc.subcore_barrier()

        for q in range(RING):
            fire_idx(q, q)
        for b in range(RING - 1):
            wait_idx(b, b)
            fire_gather(b, b)

        def body(i, carry):
            for t in range(RING):
                j = i * RING + t
                wait_gather(j, t)
                pltpu.sync_copy(gbufs[t], acc.at[ibufs[t].at[1]], add=True)
                fire_idx(j + RING, t)
                q2 = (t + RING - 1) % RING
                wait_idx(j + RING - 1, q2)
                fire_gather(j + RING - 1, q2)
            return carry

        lax.fori_loop(0, nchunks // RING, body, 0)
        for m in range(RING - 1):
            wait_gather(nchunks + m, (nchunks + m) % RING)
        wait_idx(nchunks + RING - 1, (nchunks + RING - 1) % RING)
        plsc.subcore_barrier()

        pltpu.sync_copy(acc.at[pl.ds(r0, rpt)], out_hbm.at[c, pl.ds(r0, rpt)])

    return k(h2, idx_s, zeros_tbl)


def _combine_body(p_ref, x_ref, wn_ref, ws_ref, b_ref, out_ref):
    left = p_ref[0]
    right = p_ref[1]
    cnt = jnp.max(right[:, DH - (DA - D):], axis=1, keepdims=True)
    denom = jnp.where(cnt > 0.0, cnt, 1.0)
    r = jnp.concatenate([left, right[:, :DH - (DA - D)]], axis=1) / denom
    fn = jnp.dot(r, wn_ref[...], preferred_element_type=jnp.float32)
    fx = jnp.dot(x_ref[...], ws_ref[...], preferred_element_type=jnp.float32)
    o = jnp.concatenate([fn, fx], axis=1) + b_ref[...]
    out_ref[...] = jnp.maximum(o, 0.0)


def _combine(partials, x, neighs_kernel, self_kernel, bias):
    n = x.shape[0]
    return pl.pallas_call(
        _combine_body,
        grid=(n // ROW_BLK,),
        in_specs=[
            pl.BlockSpec((NC, ROW_BLK, DH), lambda i: (0, i, 0)),
            pl.BlockSpec((ROW_BLK, D), lambda i: (i, 0)),
            pl.BlockSpec((D, D), lambda i: (0, 0)),
            pl.BlockSpec((D, D), lambda i: (0, 0)),
            pl.BlockSpec((1, 2 * D), lambda i: (0, 0)),
        ],
        out_specs=pl.BlockSpec((ROW_BLK, 2 * D), lambda i: (i, 0)),
        out_shape=jax.ShapeDtypeStruct((n, 2 * D), jnp.float32),
    )(partials, x, neighs_kernel, self_kernel, bias.reshape(1, 2 * D))


def kernel(x, edge_index, edge_weight, mlp_kernel, mlp_bias, neighs_kernel,
           self_kernel, bias):
    del edge_weight
    n_nodes = x.shape[0]
    n_edges = edge_index.shape[1]

    eps = -(-n_edges // (NS * EC * RING)) * EC * RING
    epad = NS * eps - n_edges
    t_rows = -(-(n_nodes + 1) // (NS * 8)) * (NS * 8)

    row = edge_index[0]
    col = edge_index[1]
    if epad:
        row = jnp.concatenate([row, jnp.full((epad,), n_nodes, jnp.int32)])
        col = jnp.concatenate([col, jnp.zeros((epad,), jnp.int32)])
    idx_s = jnp.stack(
        [col.reshape(NS, eps // EC, EC), row.reshape(NS, eps // EC, EC)],
        axis=2)
    idx_s = jnp.concatenate(
        [idx_s, jnp.zeros((NS, RING, 2, EC), jnp.int32)], axis=1)

    h2 = _haug(x, mlp_kernel, mlp_bias)
    zeros_tbl = jnp.zeros((t_rows, DH), jnp.float32)
    partials = _segment_sum_sc(h2, idx_s, zeros_tbl, n_nodes, t_rows, eps)
    return _combine(partials, x, neighs_kernel, self_kernel, bias)
